# Initial kernel scaffold; baseline (speedup 1.0000x reference)
import jax
import jax.numpy as jnp
from jax.experimental import pallas as pl


def kernel(X, W, T):
    raise NotImplementedError("write your pallas kernel here")


# speedup vs baseline: 10.2193x; 10.2193x over previous
import functools

import jax
import jax.numpy as jnp
from jax import lax
from jax.experimental import pallas as pl
from jax.experimental.pallas import tpu as pltpu
from jax.experimental.pallas import tpu_sc as plsc

_DX = 128
_DY = 26
_DYP = 32
_B = 4
_N = 512
_NEG = -1e30


def _emis_body(x_ref, w_ref, out_ref):
    e = jnp.dot(x_ref[0], w_ref[...], preferred_element_type=jnp.float32)
    col = lax.broadcasted_iota(jnp.int32, (_N, _DYP), 1)
    out_ref[0] = jnp.where(col >= _DY, _NEG, e)


def _compute_emis(X, Wp):
    return pl.pallas_call(
        _emis_body,
        grid=(_B,),
        in_specs=[
            pl.BlockSpec((1, _N, _DX), lambda b: (b, 0, 0)),
            pl.BlockSpec((_DX, _DYP), lambda b: (0, 0)),
        ],
        out_specs=pl.BlockSpec((1, _N, _DYP), lambda b: (b, 0, 0)),
        out_shape=jax.ShapeDtypeStruct((_B, _N, _DYP), jnp.float32),
    )(X, Wp)


_sc_mesh = plsc.VectorSubcoreMesh(core_axis_name="c", subcore_axis_name="s")


@functools.partial(
    pl.kernel,
    mesh=_sc_mesh,
    out_type=jax.ShapeDtypeStruct((_B, _N * _DYP), jnp.float32),
    scratch_types=[
        pltpu.VMEM((_N * _DYP,), jnp.float32),
        pltpu.VMEM((_DYP * _DYP,), jnp.float32),
        pltpu.VMEM((_N * _DYP,), jnp.int32),
        pltpu.VMEM((_N * _DYP,), jnp.float32),
    ],
)
def _sc_decode(emis_hbm, t_hbm, out_hbm, emis_v, t_v, bp_v, out_v):
    c = lax.axis_index("c")
    s = lax.axis_index("s")
    w = c * 2 + s

    @pl.when(s < 2)
    def _():
        pltpu.sync_copy(emis_hbm.at[w], emis_v)
        pltpu.sync_copy(t_hbm, t_v)

        def fwd_step(i, carry):
            l0, l1 = carry
            e0 = emis_v[pl.ds((i - 1) * _DYP, 16)]
            e1 = emis_v[pl.ds((i - 1) * _DYP + 16, 16)]
            acc0 = jnp.full((16,), _NEG, jnp.float32)
            acc1 = jnp.full((16,), _NEG, jnp.float32)
            bp0 = jnp.zeros((16,), jnp.int32)
            bp1 = jnp.zeros((16,), jnp.int32)
            for y0 in range(_DY):
                xe = e0[y0] if y0 < 16 else e1[y0 - 16]
                xl = l0[y0] if y0 < 16 else l1[y0 - 16]
                t0 = t_v[pl.ds(y0 * _DYP, 16)]
                t1 = t_v[pl.ds(y0 * _DYP + 16, 16)]
                c0 = (xe + t0) + xl
                c1 = (xe + t1) + xl
                m0 = c0 > acc0
                m1 = c1 > acc1
                acc0 = jnp.where(m0, c0, acc0)
                acc1 = jnp.where(m1, c1, acc1)
                bp0 = jnp.where(m0, y0, bp0)
                bp1 = jnp.where(m1, y0, bp1)
            bp_v[pl.ds(i * _DYP, 16)] = bp0
            bp_v[pl.ds(i * _DYP + 16, 16)] = bp1
            return acc0, acc1

        zeros16 = jnp.zeros((16,), jnp.float32)
        l0, l1 = lax.fori_loop(1, _N, fwd_step, (zeros16, zeros16))

        iota0 = lax.iota(jnp.int32, 16)
        iota1 = iota0 + 16

        def _butterfly(v, op):
            for sh in (8, 4, 2, 1):
                v = op(v, v.at[iota0 ^ sh].get(mode="promise_in_bounds"))
            return v

        v0 = emis_v[pl.ds((_N - 1) * _DYP, 16)] + l0
        v1 = emis_v[pl.ds((_N - 1) * _DYP + 16, 16)] + l1
        m = jnp.maximum(_butterfly(v0, jnp.maximum), _butterfly(v1, jnp.maximum))
        big = jnp.full((16,), _DYP, jnp.int32)
        a0 = jnp.where(v0 == m, iota0, big)
        a1 = jnp.where((v1 == m) & (iota1 < _DY), iota1, big)
        ans = _butterfly(jnp.minimum(a0, a1), jnp.minimum)

        one = jnp.float32(1.0)
        zero = jnp.float32(0.0)

        def write_row(i, a):
            out_v[pl.ds(i * _DYP, 16)] = jnp.where(iota0 == a, one, zero)
            out_v[pl.ds(i * _DYP + 16, 16)] = jnp.where(iota1 == a, one, zero)

        write_row(_N - 1, ans)

        fifteen = jnp.full((16,), 15, jnp.int32)

        def back_step(j, a):
            i = _N - 2 - j
            b0 = bp_v[pl.ds((i + 1) * _DYP, 16)]
            b1 = bp_v[pl.ds((i + 1) * _DYP + 16, 16)]
            g0 = b0.at[jnp.minimum(a, fifteen)].get(mode="promise_in_bounds")
            g1 = b1.at[jnp.maximum(a - 16, 0)].get(mode="promise_in_bounds")
            nxt = jnp.where(a < 16, g0, g1)
            write_row(i, nxt)
            return nxt

        lax.fori_loop(0, _N - 1, back_step, ans)

        pltpu.sync_copy(out_v, out_hbm.at[w])


def kernel(X, W, T):
    Wp = jnp.pad(W, ((0, 0), (0, _DYP - _DY)))
    Tp = jnp.pad(T, ((0, _DYP - _DY), (0, _DYP - _DY)))
    emis = _compute_emis(X, Wp).reshape(_B, _N * _DYP)
    out = _sc_decode(emis, Tp.reshape(_DYP * _DYP))
    return out.reshape(_B, _N, _DYP)[:, :, :_DY]
